# baseline (speedup 1.0000x reference)
import functools

import jax
import jax.numpy as jnp
from jax.experimental import pallas as pl
from jax.experimental.pallas import tpu as pltpu

_VMEM_LIMIT = 60 * 1024 * 1024


def _largest_divisor_at_most(n, cap):
    for d in range(min(n, cap), 0, -1):
        if n % d == 0:
            return d
    return 1


def _round8(n):
    return ((n + 7) // 8) * 8


def _conv_layer(read_src, dst_ref, w_ref, b_ref, L_out, W, Cin, Cout, nc,
                banded_out):
    CL = L_out // nc

    def compute(base):
        if Cin == 1:
            z = None
            for dy in range(3):
                slab = read_src(base + dy * W, CL + 2)
                zz = slab * w_ref[dy]
                z = zz if z is None else z + zz
        else:
            slab = read_src(base, CL + 2)
            z = jnp.dot(slab, w_ref[...],
                        preferred_element_type=jnp.float32)
        acc = (z[0:CL, 0:Cout]
               + z[1:CL + 1, Cout:2 * Cout]
               + z[2:CL + 2, 2 * Cout:3 * Cout])
        return jnp.maximum(acc + b_ref[...], 0.0)

    if not banded_out:
        def body(ci, carry):
            base = ci * CL
            dst_ref[pl.ds(base, CL), :] = compute(base)
            return carry

        jax.lax.fori_loop(0, nc, body, 0)
        return

    R = dst_ref.shape[0]
    P = min(nc, -(-2 * W // CL))
    T = min(nc - P, -(-max(0, nc * CL - R) // CL))

    def banded_store_clipped(h, base):
        for dy in range(3):
            off = dy * W
            lo = max(0, off - base)
            hi = min(CL, R - (base - off))
            if hi > lo:
                dst_ref[pl.ds(base + lo - off, hi - lo),
                        dy * Cout:(dy + 1) * Cout] = h[lo:hi]

    for ci in list(range(P)) + list(range(nc - T, nc)):
        banded_store_clipped(compute(ci * CL), ci * CL)

    def body(ci, carry):
        base = ci * CL
        h = compute(base)
        for dy in range(3):
            dst_ref[pl.ds(base - dy * W, CL),
                    dy * Cout:(dy + 1) * Cout] = h
        return carry

    if P < nc - T:
        jax.lax.fori_loop(P, nc - T, body, 0)


def _conv_stack_kernel(xf_ref, w1_ref, b1_ref, w2_ref, b2_ref, w3_ref, b3_ref,
                       o_ref, s1w, s2w, s3, *, W, H, P2pad):
    L1 = (H - 2) * W
    L2 = (H - 4) * W
    L3 = (H - 6) * W
    Hp, Wp = (H - 6) // 2, (W - 6) // 2
    C1 = s1w.shape[1] // 3
    C2 = s2w.shape[1] // 3
    C3 = s3.shape[1]

    _conv_layer(lambda s, n: xf_ref[0, pl.ds(s, n), :],
                s1w, w1_ref, b1_ref, L1, W, 1, C1,
                _largest_divisor_at_most(L1, 16), True)
    s1w[pl.ds(L2, 8), 2 * C1:3 * C1] = jnp.zeros((8, C1), jnp.float32)

    _conv_layer(lambda s, n: s1w[pl.ds(s, n), :],
                s2w, w2_ref, b2_ref, L2, W, C1, C2,
                _largest_divisor_at_most(L2, 16), True)
    s2w[pl.ds(L3, 8), 2 * C2:3 * C2] = jnp.zeros((8, C2), jnp.float32)

    _conv_layer(lambda s, n: s2w[pl.ds(s, n), :],
                s3, w3_ref, b3_ref, L3, W, C2, C3,
                _largest_divisor_at_most(L3, 16), False)

    def pbody(p, carry):
        m = None
        for i in range(2):
            for j in range(2):
                v = s3[pl.ds((2 * p + i) * W + j, Wp, 2), :]
                m = v if m is None else jnp.maximum(m, v)
        o_ref[0, pl.ds(p * Wp, Wp), :] = m
        return carry

    jax.lax.fori_loop(0, Hp, pbody, 0)

    P2 = Hp * Wp
    if P2pad > P2:
        o_ref[0, pl.ds(P2, P2pad - P2), :] = jnp.zeros(
            (P2pad - P2, C3), jnp.float32)


def _conv_stack(x_flat, w1c, b1, w2c, b2, w3c, b3, *, H, W, P2pad):
    N = x_flat.shape[0]
    Mf = H * W + 8
    xf = jnp.pad(x_flat.reshape(N, H * W, 1), ((0, 0), (0, 8), (0, 0)))
    L1, L2, L3 = (H - 2) * W, (H - 4) * W, (H - 6) * W
    C1, C2, C3 = w1c.shape[2] // 3, w2c.shape[1] // 3, w3c.shape[1] // 3
    kern = functools.partial(_conv_stack_kernel, W=W, H=H, P2pad=P2pad)
    return pl.pallas_call(
        kern,
        out_shape=jax.ShapeDtypeStruct((N, P2pad, C3), jnp.float32),
        grid=(N,),
        in_specs=[
            pl.BlockSpec((1, Mf, 1), lambda n: (n, 0, 0)),
            pl.BlockSpec(w1c.shape, lambda n: (0, 0, 0)),
            pl.BlockSpec(b1.shape, lambda n: (0, 0)),
            pl.BlockSpec(w2c.shape, lambda n: (0, 0)),
            pl.BlockSpec(b2.shape, lambda n: (0, 0)),
            pl.BlockSpec(w3c.shape, lambda n: (0, 0)),
            pl.BlockSpec(b3.shape, lambda n: (0, 0)),
        ],
        out_specs=pl.BlockSpec((1, P2pad, C3), lambda n: (n, 0, 0)),
        scratch_shapes=[
            pltpu.VMEM((_round8(L2 + 8), 3 * C1), jnp.float32),
            pltpu.VMEM((_round8(L3 + 8), 3 * C2), jnp.float32),
            pltpu.VMEM((L3, C3), jnp.float32),
        ],
        compiler_params=pltpu.CompilerParams(
            dimension_semantics=("parallel",),
            vmem_limit_bytes=_VMEM_LIMIT),
    )(xf, w1c, b1, w2c, b2, w3c, b3)


def _fc_partial_kernel(x_ref, w_ref, o_ref, acc_ref):
    k = pl.program_id(1)

    @pl.when(k == 0)
    def _():
        acc_ref[...] = jnp.zeros_like(acc_ref)

    acc_ref[...] += jnp.dot(x_ref[...], w_ref[...],
                            preferred_element_type=jnp.float32)

    @pl.when(k == pl.num_programs(1) - 1)
    def _():
        o_ref[0] = acc_ref[...]


def _fc_combine_kernel(p_ref, b1_ref, w2_ref, b2_ref, o_ref):
    h = jnp.maximum(p_ref[0] + p_ref[1] + b1_ref[...], 0.0)
    o_ref[...] = jnp.dot(h, w2_ref[...],
                         preferred_element_type=jnp.float32) + b2_ref[...]


def _fc_head(xr, w1p, b1, w2, b2):
    N, F_pad = xr.shape
    H1 = w1p.shape[1]
    C = w2.shape[1]
    assert F_pad % 256 == 0
    Kc = F_pad // 2
    tk = 128 * _largest_divisor_at_most(Kc // 128, 72)
    kpc = Kc // tk
    partials = pl.pallas_call(
        _fc_partial_kernel,
        out_shape=jax.ShapeDtypeStruct((2, N, H1), jnp.float32),
        grid=(2, kpc),
        in_specs=[
            pl.BlockSpec((N, tk), lambda c, k: (0, c * kpc + k)),
            pl.BlockSpec((tk, H1), lambda c, k: (c * kpc + k, 0)),
        ],
        out_specs=pl.BlockSpec((1, N, H1), lambda c, k: (c, 0, 0)),
        scratch_shapes=[pltpu.VMEM((N, H1), jnp.float32)],
        compiler_params=pltpu.CompilerParams(
            dimension_semantics=("parallel", "arbitrary"),
            vmem_limit_bytes=_VMEM_LIMIT),
    )(xr, w1p)
    return pl.pallas_call(
        _fc_combine_kernel,
        out_shape=jax.ShapeDtypeStruct((N, C), jnp.float32),
        grid=(1,),
        in_specs=[
            pl.BlockSpec((2, N, H1), lambda i: (0, 0, 0)),
            pl.BlockSpec(b1.shape, lambda i: (0, 0)),
            pl.BlockSpec((H1, C), lambda i: (0, 0)),
            pl.BlockSpec(b2.shape, lambda i: (0, 0)),
        ],
        out_specs=pl.BlockSpec((N, C), lambda i: (0, 0)),
        compiler_params=pltpu.CompilerParams(
            dimension_semantics=("arbitrary",),
            vmem_limit_bytes=_VMEM_LIMIT),
    )(partials, b1, w2, b2)


def _widen1(w):
    Cout = w.shape[3]
    return w.transpose(0, 2, 1, 3).reshape(3, 1, 3 * Cout)


def _widen(w):
    Cin, Cout = w.shape[2], w.shape[3]
    return w.transpose(0, 2, 1, 3).reshape(3 * Cin, 3 * Cout)


def kernel(w_conv1, b_conv1, w_conv2, b_conv2, w_conv3, b_conv3,
           w_fc1, b_fc1, w_fc2, b_fc2, w_fc1p, x):
    N, _, H, W = x.shape
    F_pad = w_fc1p.shape[0]
    C3 = w_conv3.shape[3]
    P2pad = F_pad // C3
    pooled = _conv_stack(x.reshape(N, H * W), _widen1(w_conv1), b_conv1,
                         _widen(w_conv2), b_conv2, _widen(w_conv3), b_conv3,
                         H=H, W=W, P2pad=P2pad)
    xr = pooled.reshape(N, F_pad)
    return _fc_head(xr, w_fc1p, b_fc1, w_fc2, b_fc2)
